# initial kernel scaffold (unmeasured)
import jax
import jax.numpy as jnp
from jax import lax
from jax.experimental import pallas as pl
from jax.experimental.pallas import tpu as pltpu

M = 8192
D = 2048
N_BLOCKS = 8
BLK = M // N_BLOCKS
SUB = 512
N_SUB = BLK // SUB


def _ring_coords(ridx):
    y = jnp.where(ridx < 4, 0, 1)
    z = jnp.where(ridx < 4, ridx, 7 - ridx)
    return y, z


def kernel(partial, resid, gamma):
    def body(partial_ref, resid_ref, gamma_ref, out_ref,
             peer_buf, p_chunk, r_chunk, o_chunk,
             x_send_sem, x_recv_sem,
             ag_send_sem, ag_recv_sems,
             local_sems):
        my_x = lax.axis_index("x")
        my_y = lax.axis_index("y")
        my_z = lax.axis_index("z")
        r = jnp.where(my_y == 0, my_z, 7 - my_z)
        blk_start = r * BLK

        rdma_x = pltpu.make_async_remote_copy(
            src_ref=partial_ref.at[0, pl.ds(blk_start, BLK), :],
            dst_ref=peer_buf,
            send_sem=x_send_sem,
            recv_sem=x_recv_sem,
            device_id=(1 - my_x, my_y, my_z),
            device_id_type=pl.DeviceIdType.MESH,
        )
        rdma_x.start()
        rdma_x.wait()

        for s in range(N_SUB):
            row0 = blk_start + s * SUB
            cp_p = pltpu.make_async_copy(
                partial_ref.at[0, pl.ds(row0, SUB), :], p_chunk,
                local_sems.at[0])
            cp_r = pltpu.make_async_copy(
                resid_ref.at[pl.ds(row0, SUB), :], r_chunk,
                local_sems.at[1])
            cp_p.start()
            cp_r.start()
            cp_p.wait()
            cp_r.wait()
            y = (p_chunk[...] + peer_buf[pl.ds(s * SUB, SUB), :]
                 + r_chunk[...])
            rms = jnp.sqrt(jnp.mean(y * y, axis=-1, keepdims=True) + 1e-6)
            o_chunk[...] = y / rms * gamma_ref[...]
            cp_o = pltpu.make_async_copy(
                o_chunk, out_ref.at[pl.ds(row0, SUB), :], local_sems.at[2])
            cp_o.start()
            cp_o.wait()

        rr = lax.rem(r + 1, N_BLOCKS)
        right_y, right_z = _ring_coords(rr)
        for h in range(N_BLOCKS - 1):
            send_blk = lax.rem(r - h + N_BLOCKS, N_BLOCKS)
            send_off = send_blk * BLK
            rdma = pltpu.make_async_remote_copy(
                src_ref=out_ref.at[pl.ds(send_off, BLK), :],
                dst_ref=out_ref.at[pl.ds(send_off, BLK), :],
                send_sem=ag_send_sem,
                recv_sem=ag_recv_sems.at[h],
                device_id=(my_x, right_y, right_z),
                device_id_type=pl.DeviceIdType.MESH,
            )
            rdma.start()
            rdma.wait()

    out_shape = jax.ShapeDtypeStruct((M, D), jnp.float32)
    return pl.pallas_call(
        body,
        out_shape=out_shape,
        in_specs=[
            pl.BlockSpec(memory_space=pltpu.ANY),
            pl.BlockSpec(memory_space=pltpu.ANY),
            pl.BlockSpec(memory_space=pltpu.VMEM),
        ],
        out_specs=pl.BlockSpec(memory_space=pltpu.ANY),
        scratch_shapes=[
            pltpu.VMEM((BLK, D), jnp.float32),
            pltpu.VMEM((SUB, D), jnp.float32),
            pltpu.VMEM((SUB, D), jnp.float32),
            pltpu.VMEM((SUB, D), jnp.float32),
            pltpu.SemaphoreType.DMA,
            pltpu.SemaphoreType.DMA,
            pltpu.SemaphoreType.DMA,
            pltpu.SemaphoreType.DMA((N_BLOCKS,)),
            pltpu.SemaphoreType.DMA((3,)),
        ],
    )(partial, resid, gamma)


# baseline (device time: 802897 ns/iter reference)
import jax
import jax.numpy as jnp
from jax import lax
from jax.experimental import pallas as pl
from jax.experimental.pallas import tpu as pltpu

M = 8192
D = 2048
N_BLOCKS = 8
BLK = M // N_BLOCKS
SUB = 512
N_SUB = BLK // SUB


def _ring_coords(ridx):
    y = jnp.where(ridx < 4, 0, 1)
    z = jnp.where(ridx < 4, ridx, 7 - ridx)
    return y, z


def kernel(partial, resid, gamma):
    def body(partial_ref, resid_ref, gamma_ref, out_ref,
             peer_buf, p_chunk, r_chunk, o_chunk,
             x_send_sem, x_recv_sem,
             ag_send_sem, ag_recv_sems,
             local_sems):
        my_x = lax.axis_index("x")
        my_y = lax.axis_index("y")
        my_z = lax.axis_index("z")
        r = jnp.where(my_y == 0, my_z, 7 - my_z)
        blk_start = r * BLK

        rdma_x = pltpu.make_async_remote_copy(
            src_ref=partial_ref.at[0, pl.ds(blk_start, BLK), :],
            dst_ref=peer_buf,
            send_sem=x_send_sem,
            recv_sem=x_recv_sem,
            device_id=(1 - my_x, my_y, my_z),
            device_id_type=pl.DeviceIdType.MESH,
        )
        rdma_x.start()
        rdma_x.wait()

        for s in range(N_SUB):
            row0 = blk_start + s * SUB
            cp_p = pltpu.make_async_copy(
                partial_ref.at[0, pl.ds(row0, SUB), :], p_chunk,
                local_sems.at[0])
            cp_r = pltpu.make_async_copy(
                resid_ref.at[pl.ds(row0, SUB), :], r_chunk,
                local_sems.at[1])
            cp_p.start()
            cp_r.start()
            cp_p.wait()
            cp_r.wait()
            y = (p_chunk[...] + peer_buf[pl.ds(s * SUB, SUB), :]
                 + r_chunk[...])
            rms = jnp.sqrt(jnp.mean(y * y, axis=-1, keepdims=True) + 1e-6)
            o_chunk[...] = y / rms * gamma_ref[...]
            cp_o = pltpu.make_async_copy(
                o_chunk, out_ref.at[pl.ds(row0, SUB), :], local_sems.at[2])
            cp_o.start()
            cp_o.wait()

        rr = lax.rem(r + 1, N_BLOCKS)
        right_y, right_z = _ring_coords(rr)
        for h in range(N_BLOCKS - 1):
            send_blk = lax.rem(r - h + N_BLOCKS, N_BLOCKS)
            send_off = send_blk * BLK
            rdma = pltpu.make_async_remote_copy(
                src_ref=out_ref.at[pl.ds(send_off, BLK), :],
                dst_ref=out_ref.at[pl.ds(send_off, BLK), :],
                send_sem=ag_send_sem,
                recv_sem=ag_recv_sems.at[h],
                device_id=(my_x, right_y, right_z),
                device_id_type=pl.DeviceIdType.MESH,
            )
            rdma.start()
            rdma.wait()

    out_shape = jax.ShapeDtypeStruct((M, D), jnp.float32)
    return pl.pallas_call(
        body,
        out_shape=out_shape,
        in_specs=[
            pl.BlockSpec(memory_space=pl.ANY),
            pl.BlockSpec(memory_space=pl.ANY),
            pl.BlockSpec(memory_space=pltpu.VMEM),
        ],
        out_specs=pl.BlockSpec(memory_space=pl.ANY),
        scratch_shapes=[
            pltpu.VMEM((BLK, D), jnp.float32),
            pltpu.VMEM((SUB, D), jnp.float32),
            pltpu.VMEM((SUB, D), jnp.float32),
            pltpu.VMEM((SUB, D), jnp.float32),
            pltpu.SemaphoreType.DMA,
            pltpu.SemaphoreType.DMA,
            pltpu.SemaphoreType.DMA,
            pltpu.SemaphoreType.DMA((N_BLOCKS,)),
            pltpu.SemaphoreType.DMA((3,)),
        ],
    )(partial, resid, gamma)


# device time: 489276 ns/iter; 1.6410x vs baseline; 1.6410x over previous
import jax
import jax.numpy as jnp
from jax import lax
from jax.experimental import pallas as pl
from jax.experimental.pallas import tpu as pltpu

M = 8192
D = 2048
N_BLOCKS = 8
BLK = M // N_BLOCKS
SUB = 512
N_SUB = BLK // SUB


def _ring_coords(ridx):
    y = jnp.where(ridx < 4, 0, 1)
    z = jnp.where(ridx < 4, ridx, 7 - ridx)
    return y, z


def kernel(partial, resid, gamma):
    def body(partial_ref, resid_ref, gamma_ref, out_ref,
             peer_buf, p_chunk, r_chunk, o_chunk,
             x_send_sem, x_recv_sem,
             ag_send_sem, ag_recv_sems,
             bwd_send_sem, bwd_recv_sems,
             local_sems):
        my_x = lax.axis_index("x")
        my_y = lax.axis_index("y")
        my_z = lax.axis_index("z")
        r = jnp.where(my_y == 0, my_z, 7 - my_z)
        blk_start = r * BLK

        rdma_x = pltpu.make_async_remote_copy(
            src_ref=partial_ref.at[0, pl.ds(blk_start, BLK), :],
            dst_ref=peer_buf,
            send_sem=x_send_sem,
            recv_sem=x_recv_sem,
            device_id=(1 - my_x, my_y, my_z),
            device_id_type=pl.DeviceIdType.MESH,
        )
        rdma_x.start()
        rdma_x.wait()

        for s in range(N_SUB):
            row0 = blk_start + s * SUB
            cp_p = pltpu.make_async_copy(
                partial_ref.at[0, pl.ds(row0, SUB), :], p_chunk,
                local_sems.at[0])
            cp_r = pltpu.make_async_copy(
                resid_ref.at[pl.ds(row0, SUB), :], r_chunk,
                local_sems.at[1])
            cp_p.start()
            cp_r.start()
            cp_p.wait()
            cp_r.wait()
            y = (p_chunk[...] + peer_buf[pl.ds(s * SUB, SUB), :]
                 + r_chunk[...])
            rms = jnp.sqrt(jnp.mean(y * y, axis=-1, keepdims=True) + 1e-6)
            o_chunk[...] = y / rms * gamma_ref[...]
            cp_o = pltpu.make_async_copy(
                o_chunk, out_ref.at[pl.ds(row0, SUB), :], local_sems.at[2])
            cp_o.start()
            cp_o.wait()

        rr = lax.rem(r + 1, N_BLOCKS)
        rl = lax.rem(r - 1 + N_BLOCKS, N_BLOCKS)
        right_y, right_z = _ring_coords(rr)
        left_y, left_z = _ring_coords(rl)
        HALF = BLK // 2
        for h in range(N_BLOCKS - 1):
            fwd_blk = lax.rem(r - h + N_BLOCKS, N_BLOCKS)
            bwd_blk = lax.rem(r + h, N_BLOCKS)
            fwd_off = fwd_blk * BLK
            bwd_off = bwd_blk * BLK + HALF
            rdma_f = pltpu.make_async_remote_copy(
                src_ref=out_ref.at[pl.ds(fwd_off, HALF), :],
                dst_ref=out_ref.at[pl.ds(fwd_off, HALF), :],
                send_sem=ag_send_sem,
                recv_sem=ag_recv_sems.at[h],
                device_id=(my_x, right_y, right_z),
                device_id_type=pl.DeviceIdType.MESH,
            )
            rdma_b = pltpu.make_async_remote_copy(
                src_ref=out_ref.at[pl.ds(bwd_off, HALF), :],
                dst_ref=out_ref.at[pl.ds(bwd_off, HALF), :],
                send_sem=bwd_send_sem,
                recv_sem=bwd_recv_sems.at[h],
                device_id=(my_x, left_y, left_z),
                device_id_type=pl.DeviceIdType.MESH,
            )
            rdma_f.start()
            rdma_b.start()
            rdma_f.wait()
            rdma_b.wait()

    out_shape = jax.ShapeDtypeStruct((M, D), jnp.float32)
    return pl.pallas_call(
        body,
        out_shape=out_shape,
        in_specs=[
            pl.BlockSpec(memory_space=pl.ANY),
            pl.BlockSpec(memory_space=pl.ANY),
            pl.BlockSpec(memory_space=pltpu.VMEM),
        ],
        out_specs=pl.BlockSpec(memory_space=pl.ANY),
        scratch_shapes=[
            pltpu.VMEM((BLK, D), jnp.float32),
            pltpu.VMEM((SUB, D), jnp.float32),
            pltpu.VMEM((SUB, D), jnp.float32),
            pltpu.VMEM((SUB, D), jnp.float32),
            pltpu.SemaphoreType.DMA,
            pltpu.SemaphoreType.DMA,
            pltpu.SemaphoreType.DMA,
            pltpu.SemaphoreType.DMA((N_BLOCKS,)),
            pltpu.SemaphoreType.DMA,
            pltpu.SemaphoreType.DMA((N_BLOCKS,)),
            pltpu.SemaphoreType.DMA((3,)),
        ],
    )(partial, resid, gamma)


# device time: 480192 ns/iter; 1.6720x vs baseline; 1.0189x over previous
import jax
import jax.numpy as jnp
from jax import lax
from jax.experimental import pallas as pl
from jax.experimental.pallas import tpu as pltpu

M = 8192
D = 2048
N_BLOCKS = 8
BLK = M // N_BLOCKS
HALF = BLK // 2
N_HOP = N_BLOCKS - 1


def _ring_coords(ridx):
    y = jnp.where(ridx < 4, 0, 1)
    z = jnp.where(ridx < 4, ridx, 7 - ridx)
    return y, z


def kernel(partial, resid, gamma):
    def body(partial_ref, resid_ref, gamma_ref, out_ref,
             peer_buf, p_chunks, r_chunks, o_chunks,
             p1_send, p1_recv,
             f_send, b_send, f_recv, b_recv,
             local_sems):
        my_x = lax.axis_index("x")
        my_y = lax.axis_index("y")
        my_z = lax.axis_index("z")
        r = jnp.where(my_y == 0, my_z, 7 - my_z)
        bb0 = r * BLK

        rr = lax.rem(r + 1, N_BLOCKS)
        rl = lax.rem(r - 1 + N_BLOCKS, N_BLOCKS)
        right_y, right_z = _ring_coords(rr)
        left_y, left_z = _ring_coords(rl)
        right_dev = (my_x, right_y, right_z)
        left_dev = (my_x, left_y, left_z)
        xpeer_dev = (1 - my_x, my_y, my_z)

        p1 = []
        for k in range(2):
            off = bb0 + k * HALF
            rdma = pltpu.make_async_remote_copy(
                src_ref=partial_ref.at[0, pl.ds(off, HALF), :],
                dst_ref=peer_buf.at[pl.ds(k * HALF, HALF), :],
                send_sem=p1_send.at[k],
                recv_sem=p1_recv.at[k],
                device_id=xpeer_dev,
                device_id_type=pl.DeviceIdType.MESH,
            )
            rdma.start()
            p1.append(rdma)

        cps = []
        for k in range(2):
            off = bb0 + k * HALF
            cp_p = pltpu.make_async_copy(
                partial_ref.at[0, pl.ds(off, HALF), :], p_chunks.at[k],
                local_sems.at[2 * k])
            cp_r = pltpu.make_async_copy(
                resid_ref.at[pl.ds(off, HALF), :], r_chunks.at[k],
                local_sems.at[2 * k + 1])
            cp_p.start()
            cp_r.start()
            cps.append((cp_p, cp_r, off))

        def ring_rdma(off, send_sem, recv_sem, dev):
            return pltpu.make_async_remote_copy(
                src_ref=out_ref.at[pl.ds(off, HALF), :],
                dst_ref=out_ref.at[pl.ds(off, HALF), :],
                send_sem=send_sem, recv_sem=recv_sem,
                device_id=dev, device_id_type=pl.DeviceIdType.MESH,
            )

        hop0 = [None, None]
        for k in range(2):
            cp_p, cp_r, off = cps[k]
            p1[k].wait()
            cp_p.wait()
            cp_r.wait()
            y = (p_chunks[k] + peer_buf[k * HALF:(k + 1) * HALF, :]
                 + r_chunks[k])
            rms = jnp.sqrt(jnp.mean(y * y, axis=-1, keepdims=True) + 1e-6)
            o_chunks[k] = y / rms * gamma_ref[...]
            cp_o = pltpu.make_async_copy(
                o_chunks.at[k], out_ref.at[pl.ds(off, HALF), :],
                local_sems.at[4 + k])
            cp_o.start()
            cp_o.wait()
            if k == 0:
                df = ring_rdma(bb0, f_send, f_recv.at[0], right_dev)
                df.start()
                hop0[0] = df
            else:
                db = ring_rdma(bb0 + HALF, b_send, b_recv.at[0], left_dev)
                db.start()
                hop0[1] = db

        cur = tuple(hop0)
        for h in range(N_HOP):
            df, db = cur
            df.wait()
            db.wait()
            if h + 1 < N_HOP:
                fb = lax.rem(r + N_BLOCKS - (h + 1), N_BLOCKS) * BLK
                bbk = lax.rem(r + h + 1, N_BLOCKS) * BLK
                ndf = ring_rdma(fb, f_send, f_recv.at[h + 1], right_dev)
                ndb = ring_rdma(bbk + HALF, b_send, b_recv.at[h + 1],
                                left_dev)
                ndf.start()
                ndb.start()
                cur = (ndf, ndb)

    out_shape = jax.ShapeDtypeStruct((M, D), jnp.float32)
    return pl.pallas_call(
        body,
        out_shape=out_shape,
        in_specs=[
            pl.BlockSpec(memory_space=pl.ANY),
            pl.BlockSpec(memory_space=pl.ANY),
            pl.BlockSpec(memory_space=pltpu.VMEM),
        ],
        out_specs=pl.BlockSpec(memory_space=pl.ANY),
        scratch_shapes=[
            pltpu.VMEM((BLK, D), jnp.float32),
            pltpu.VMEM((2, HALF, D), jnp.float32),
            pltpu.VMEM((2, HALF, D), jnp.float32),
            pltpu.VMEM((2, HALF, D), jnp.float32),
            pltpu.SemaphoreType.DMA((2,)),
            pltpu.SemaphoreType.DMA((2,)),
            pltpu.SemaphoreType.DMA,
            pltpu.SemaphoreType.DMA,
            pltpu.SemaphoreType.DMA((N_HOP,)),
            pltpu.SemaphoreType.DMA((N_HOP,)),
            pltpu.SemaphoreType.DMA((6,)),
        ],
        compiler_params=pltpu.CompilerParams(
            vmem_limit_bytes=64 * 1024 * 1024),
    )(partial, resid, gamma)


# device time: 467559 ns/iter; 1.7172x vs baseline; 1.0270x over previous
import jax
import jax.numpy as jnp
from jax import lax
from jax.experimental import pallas as pl
from jax.experimental.pallas import tpu as pltpu

M = 8192
D = 2048
N_BLOCKS = 8
BLK = M // N_BLOCKS
HALF = BLK // 2
N_HOP = N_BLOCKS - 1


def _ring_coords(ridx):
    y = jnp.where(ridx < 4, 0, 1)
    z = jnp.where(ridx < 4, ridx, 7 - ridx)
    return y, z


def kernel(partial, resid, gamma):
    def body(partial_ref, resid_ref, gamma_ref, out_ref,
             peer_buf, p_chunks, r_chunks, o_chunks,
             p1_send, p1_recv,
             f0_send, f1_send, b0_send, b1_send,
             f0_recv, f1_recv, b0_recv, b1_recv,
             local_sems):
        my_x = lax.axis_index("x")
        my_y = lax.axis_index("y")
        my_z = lax.axis_index("z")
        r = jnp.where(my_y == 0, my_z, 7 - my_z)
        bb0 = r * BLK

        rr = lax.rem(r + 1, N_BLOCKS)
        rl = lax.rem(r - 1 + N_BLOCKS, N_BLOCKS)
        right_y, right_z = _ring_coords(rr)
        left_y, left_z = _ring_coords(rl)
        right_dev = (my_x, right_y, right_z)
        left_dev = (my_x, left_y, left_z)
        xpeer_dev = (1 - my_x, my_y, my_z)

        p1 = []
        for k in range(2):
            off = bb0 + k * HALF
            rdma = pltpu.make_async_remote_copy(
                src_ref=partial_ref.at[0, pl.ds(off, HALF), :],
                dst_ref=peer_buf.at[pl.ds(k * HALF, HALF), :],
                send_sem=p1_send.at[k],
                recv_sem=p1_recv.at[k],
                device_id=xpeer_dev,
                device_id_type=pl.DeviceIdType.MESH,
            )
            rdma.start()
            p1.append(rdma)

        cps = []
        for k in range(2):
            off = bb0 + k * HALF
            cp_p = pltpu.make_async_copy(
                partial_ref.at[0, pl.ds(off, HALF), :], p_chunks.at[k],
                local_sems.at[2 * k])
            cp_r = pltpu.make_async_copy(
                resid_ref.at[pl.ds(off, HALF), :], r_chunks.at[k],
                local_sems.at[2 * k + 1])
            cp_p.start()
            cp_r.start()
            cps.append((cp_p, cp_r, off))

        QTR = HALF // 2

        def ring_rdma(off, send_sem, recv_sem, dev):
            return pltpu.make_async_remote_copy(
                src_ref=out_ref.at[pl.ds(off, QTR), :],
                dst_ref=out_ref.at[pl.ds(off, QTR), :],
                send_sem=send_sem, recv_sem=recv_sem,
                device_id=dev, device_id_type=pl.DeviceIdType.MESH,
            )

        hop0 = [None, None, None, None]
        for k in range(2):
            cp_p, cp_r, off = cps[k]
            p1[k].wait()
            cp_p.wait()
            cp_r.wait()
            y = (p_chunks[k] + peer_buf[k * HALF:(k + 1) * HALF, :]
                 + r_chunks[k])
            rms = jnp.sqrt(jnp.mean(y * y, axis=-1, keepdims=True) + 1e-6)
            o_chunks[k] = y / rms * gamma_ref[...]
            cp_o = pltpu.make_async_copy(
                o_chunks.at[k], out_ref.at[pl.ds(off, HALF), :],
                local_sems.at[4 + k])
            cp_o.start()
            cp_o.wait()
            if k == 0:
                d0 = ring_rdma(bb0, f0_send, f0_recv.at[0], right_dev)
                d1 = ring_rdma(bb0 + QTR, f1_send, f1_recv.at[0],
                               right_dev)
                d0.start()
                d1.start()
                hop0[0], hop0[1] = d0, d1
            else:
                d0 = ring_rdma(bb0 + HALF, b0_send, b0_recv.at[0],
                               left_dev)
                d1 = ring_rdma(bb0 + HALF + QTR, b1_send, b1_recv.at[0],
                               left_dev)
                d0.start()
                d1.start()
                hop0[2], hop0[3] = d0, d1

        cur = list(hop0)
        sub_off = (0, QTR, HALF, HALF + QTR)
        sub_sems = ((f0_send, f0_recv), (f1_send, f1_recv),
                    (b0_send, b0_recv), (b1_send, b1_recv))
        for h in range(N_HOP):
            fb = lax.rem(r + N_BLOCKS - (h + 1), N_BLOCKS) * BLK
            bbk = lax.rem(r + h + 1, N_BLOCKS) * BLK
            nxt_blk = (fb, fb, bbk, bbk)
            for s in range(4):
                cur[s].wait()
                if h + 1 < N_HOP:
                    snd, rcv = sub_sems[s]
                    dev = right_dev if s < 2 else left_dev
                    nd = ring_rdma(nxt_blk[s] + sub_off[s], snd,
                                   rcv.at[h + 1], dev)
                    nd.start()
                    cur[s] = nd

    out_shape = jax.ShapeDtypeStruct((M, D), jnp.float32)
    return pl.pallas_call(
        body,
        out_shape=out_shape,
        in_specs=[
            pl.BlockSpec(memory_space=pl.ANY),
            pl.BlockSpec(memory_space=pl.ANY),
            pl.BlockSpec(memory_space=pltpu.VMEM),
        ],
        out_specs=pl.BlockSpec(memory_space=pl.ANY),
        scratch_shapes=[
            pltpu.VMEM((BLK, D), jnp.float32),
            pltpu.VMEM((2, HALF, D), jnp.float32),
            pltpu.VMEM((2, HALF, D), jnp.float32),
            pltpu.VMEM((2, HALF, D), jnp.float32),
            pltpu.SemaphoreType.DMA((2,)),
            pltpu.SemaphoreType.DMA((2,)),
            pltpu.SemaphoreType.DMA,
            pltpu.SemaphoreType.DMA,
            pltpu.SemaphoreType.DMA,
            pltpu.SemaphoreType.DMA,
            pltpu.SemaphoreType.DMA((N_HOP,)),
            pltpu.SemaphoreType.DMA((N_HOP,)),
            pltpu.SemaphoreType.DMA((N_HOP,)),
            pltpu.SemaphoreType.DMA((N_HOP,)),
            pltpu.SemaphoreType.DMA((6,)),
        ],
        compiler_params=pltpu.CompilerParams(
            vmem_limit_bytes=64 * 1024 * 1024),
    )(partial, resid, gamma)


# device time: 443588 ns/iter; 1.8100x vs baseline; 1.0540x over previous
import jax
import jax.numpy as jnp
from jax import lax
from jax.experimental import pallas as pl
from jax.experimental.pallas import tpu as pltpu

M = 8192
D = 2048
N_BLOCKS = 8
BLK = M // N_BLOCKS
HALF = BLK // 2
N_HOP = N_BLOCKS - 1


def _ring_coords(ridx):
    y = jnp.where(ridx < 4, 0, 1)
    z = jnp.where(ridx < 4, ridx, 7 - ridx)
    return y, z


def kernel(partial, resid, gamma):
    def body(partial_ref, resid_ref, gamma_ref, out_ref,
             peer_buf, p_chunks, r_chunks, o_chunks,
             p1_send, p1_recv,
             f0_send, f1_send, b0_send, b1_send,
             f0_recv, f1_recv, b0_recv, b1_recv,
             local_sems):
        my_x = lax.axis_index("x")
        my_y = lax.axis_index("y")
        my_z = lax.axis_index("z")
        r = jnp.where(my_y == 0, my_z, 7 - my_z)
        bb0 = r * BLK

        rr = lax.rem(r + 1, N_BLOCKS)
        rl = lax.rem(r - 1 + N_BLOCKS, N_BLOCKS)
        right_y, right_z = _ring_coords(rr)
        left_y, left_z = _ring_coords(rl)
        right_dev = (my_x, right_y, right_z)
        left_dev = (my_x, left_y, left_z)
        xpeer_dev = (1 - my_x, my_y, my_z)

        QTR4 = BLK // 4
        ORDER = (0, 2, 1, 3)

        p1 = [None] * 4
        for i, q in enumerate(ORDER):
            off = bb0 + q * QTR4
            rdma = pltpu.make_async_remote_copy(
                src_ref=partial_ref.at[0, pl.ds(off, QTR4), :],
                dst_ref=peer_buf.at[pl.ds(q * QTR4, QTR4), :],
                send_sem=p1_send.at[i],
                recv_sem=p1_recv.at[i],
                device_id=xpeer_dev,
                device_id_type=pl.DeviceIdType.MESH,
            )
            rdma.start()
            p1[q] = rdma

        cps = [None] * 4
        for q in range(4):
            off = bb0 + q * QTR4
            cp_p = pltpu.make_async_copy(
                partial_ref.at[0, pl.ds(off, QTR4), :], p_chunks.at[q],
                local_sems.at[2 * q])
            cp_r = pltpu.make_async_copy(
                resid_ref.at[pl.ds(off, QTR4), :], r_chunks.at[q],
                local_sems.at[2 * q + 1])
            cp_p.start()
            cp_r.start()
            cps[q] = (cp_p, cp_r, off)

        QTR = HALF // 2

        def ring_rdma(off, send_sem, recv_sem, dev):
            return pltpu.make_async_remote_copy(
                src_ref=out_ref.at[pl.ds(off, QTR), :],
                dst_ref=out_ref.at[pl.ds(off, QTR), :],
                send_sem=send_sem, recv_sem=recv_sem,
                device_id=dev, device_id_type=pl.DeviceIdType.MESH,
            )

        sub_off = (0, QTR, HALF, HALF + QTR)
        sub_sems = ((f0_send, f0_recv), (f1_send, f1_recv),
                    (b0_send, b0_recv), (b1_send, b1_recv))
        hop0 = [None, None, None, None]
        for q in ORDER:
            cp_p, cp_r, off = cps[q]
            p1[q].wait()
            cp_p.wait()
            cp_r.wait()
            y = (p_chunks[q] + peer_buf[q * QTR4:(q + 1) * QTR4, :]
                 + r_chunks[q])
            rms = jnp.sqrt(jnp.mean(y * y, axis=-1, keepdims=True) + 1e-6)
            o_chunks[q] = y / rms * gamma_ref[...]
            cp_o = pltpu.make_async_copy(
                o_chunks.at[q], out_ref.at[pl.ds(off, QTR4), :],
                local_sems.at[8 + q])
            cp_o.start()
            cp_o.wait()
            snd, rcv = sub_sems[q]
            dev = right_dev if q < 2 else left_dev
            d = ring_rdma(bb0 + sub_off[q], snd, rcv.at[0], dev)
            d.start()
            hop0[q] = d

        cur = list(hop0)
        for h in range(N_HOP):
            fb = lax.rem(r + N_BLOCKS - (h + 1), N_BLOCKS) * BLK
            bbk = lax.rem(r + h + 1, N_BLOCKS) * BLK
            nxt_blk = (fb, fb, bbk, bbk)
            for s in range(4):
                cur[s].wait()
                if h + 1 < N_HOP:
                    snd, rcv = sub_sems[s]
                    dev = right_dev if s < 2 else left_dev
                    nd = ring_rdma(nxt_blk[s] + sub_off[s], snd,
                                   rcv.at[h + 1], dev)
                    nd.start()
                    cur[s] = nd

    out_shape = jax.ShapeDtypeStruct((M, D), jnp.float32)
    return pl.pallas_call(
        body,
        out_shape=out_shape,
        in_specs=[
            pl.BlockSpec(memory_space=pl.ANY),
            pl.BlockSpec(memory_space=pl.ANY),
            pl.BlockSpec(memory_space=pltpu.VMEM),
        ],
        out_specs=pl.BlockSpec(memory_space=pl.ANY),
        scratch_shapes=[
            pltpu.VMEM((BLK, D), jnp.float32),
            pltpu.VMEM((4, BLK // 4, D), jnp.float32),
            pltpu.VMEM((4, BLK // 4, D), jnp.float32),
            pltpu.VMEM((4, BLK // 4, D), jnp.float32),
            pltpu.SemaphoreType.DMA((4,)),
            pltpu.SemaphoreType.DMA((4,)),
            pltpu.SemaphoreType.DMA,
            pltpu.SemaphoreType.DMA,
            pltpu.SemaphoreType.DMA,
            pltpu.SemaphoreType.DMA,
            pltpu.SemaphoreType.DMA((N_HOP,)),
            pltpu.SemaphoreType.DMA((N_HOP,)),
            pltpu.SemaphoreType.DMA((N_HOP,)),
            pltpu.SemaphoreType.DMA((N_HOP,)),
            pltpu.SemaphoreType.DMA((12,)),
        ],
        compiler_params=pltpu.CompilerParams(
            vmem_limit_bytes=64 * 1024 * 1024),
    )(partial, resid, gamma)
